# baseline (speedup 1.0000x reference)
import functools

import jax
import jax.numpy as jnp
from jax import lax
from jax.experimental import pallas as pl
from jax.experimental.pallas import tpu as pltpu
from jax.experimental.pallas import tpu_sc as plsc

VOCAB = 1000000
EMB_DIM = 64
NUM_CLASS = 38
BATCH = 16384

NUM_CORES = 2
NUM_SUBCORES = 16
NUM_WORKERS = NUM_CORES * NUM_SUBCORES
B_PER_W = BATCH // NUM_WORKERS
CHUNK = 128
N_CHUNKS = B_PER_W // CHUNK


def _gather_body(emb_hbm, node_hbm, out_hbm, idx_v, rows_v, sem):
    wid = lax.axis_index("s") * NUM_CORES + lax.axis_index("c")
    pltpu.sync_copy(node_hbm.at[wid], idx_v)
    copies = []
    for j in range(N_CHUNKS):
        copies.append(
            pltpu.make_async_copy(emb_hbm.at[idx_v.at[j]], rows_v.at[j], sem)
        )
        copies[-1].start()
    for c in copies:
        c.wait()
    pltpu.sync_copy(rows_v, out_hbm.at[wid])


@jax.jit
def _sc_gather(emb, node3d):
    mesh = plsc.VectorSubcoreMesh(core_axis_name="c", subcore_axis_name="s")
    return pl.kernel(
        _gather_body,
        out_type=jax.ShapeDtypeStruct(
            (NUM_WORKERS, N_CHUNKS, CHUNK, EMB_DIM), jnp.float32
        ),
        mesh=mesh,
        scratch_types=[
            pltpu.VMEM((N_CHUNKS, CHUNK), jnp.int32),
            pltpu.VMEM((N_CHUNKS, CHUNK, EMB_DIM), jnp.float32),
            pltpu.SemaphoreType.DMA,
        ],
        compiler_params=pltpu.CompilerParams(use_tc_tiling_on_sc=False),
    )(emb, node3d)


def _linear_body(x_ref, w_ref, b_ref, o_ref):
    o_ref[...] = (
        jnp.dot(x_ref[...], w_ref[...], preferred_element_type=jnp.float32)
        + b_ref[...]
    )


@jax.jit
def _tc_linear(x, w_t, b2d):
    block = 2048
    grid = (BATCH // block,)
    return pl.pallas_call(
        _linear_body,
        grid=grid,
        in_specs=[
            pl.BlockSpec((block, EMB_DIM), lambda i: (i, 0)),
            pl.BlockSpec((EMB_DIM, NUM_CLASS), lambda i: (0, 0)),
            pl.BlockSpec((1, NUM_CLASS), lambda i: (0, 0)),
        ],
        out_specs=pl.BlockSpec((block, NUM_CLASS), lambda i: (i, 0)),
        out_shape=jax.ShapeDtypeStruct((BATCH, NUM_CLASS), jnp.float32),
    )(x, w_t, b2d)


def kernel(node, emb, fc_w, fc_b):
    node3d = node.astype(jnp.int32).reshape(NUM_WORKERS, N_CHUNKS, CHUNK)
    gathered = _sc_gather(emb, node3d).reshape(BATCH, EMB_DIM)
    return _tc_linear(gathered, fc_w.T, fc_b.reshape(1, NUM_CLASS))


# speedup vs baseline: 1.9379x; 1.9379x over previous
import functools

import jax
import jax.numpy as jnp
from jax import lax
from jax.experimental import pallas as pl
from jax.experimental.pallas import tpu as pltpu
from jax.experimental.pallas import tpu_sc as plsc

VOCAB = 1000000
EMB_DIM = 64
NUM_CLASS = 38
BATCH = 16384

NUM_CORES = 2
NUM_SUBCORES = 16
NUM_WORKERS = NUM_CORES * NUM_SUBCORES
B_PER_W = BATCH // NUM_WORKERS
LANES = 16
N_GROUPS = B_PER_W // LANES


def _gather_body(emb_hbm, idx_hbm, out_hbm, idx_v, rows_v, sem):
    wid = lax.axis_index("s") * NUM_CORES + lax.axis_index("c")
    pltpu.sync_copy(idx_hbm.at[wid], idx_v)

    lanes = lax.iota(jnp.int32, LANES)

    def fire(g, _):
        vec = idx_v[g, :]
        for k in range(LANES):
            t = lax.reduce_sum(jnp.where(lanes == k, vec, 0), axes=(0,))
            pltpu.async_copy(
                emb_hbm.at[t], rows_v.at[g * LANES + k], sem
            ).start()
        return 0

    lax.fori_loop(0, N_GROUPS, fire, 0)

    def drain(i, _):
        pltpu.make_async_copy(emb_hbm.at[0], rows_v.at[i], sem).wait()
        return 0

    lax.fori_loop(0, B_PER_W, drain, 0)

    pltpu.sync_copy(rows_v, out_hbm.at[wid])


@jax.jit
def _sc_gather(emb, idx3d):
    mesh = plsc.VectorSubcoreMesh(core_axis_name="c", subcore_axis_name="s")
    return pl.kernel(
        _gather_body,
        out_type=jax.ShapeDtypeStruct(
            (NUM_WORKERS, B_PER_W, EMB_DIM), jnp.float32
        ),
        mesh=mesh,
        scratch_types=[
            pltpu.VMEM((N_GROUPS, LANES), jnp.int32),
            pltpu.VMEM((B_PER_W, EMB_DIM), jnp.float32),
            pltpu.SemaphoreType.DMA,
        ],
        compiler_params=pltpu.CompilerParams(needs_layout_passes=False),
    )(emb, idx3d)


def _linear_body(x_ref, w_ref, b_ref, o_ref):
    o_ref[...] = (
        jnp.dot(x_ref[...], w_ref[...], preferred_element_type=jnp.float32)
        + b_ref[...]
    )


@jax.jit
def _tc_linear(x, w_t, b2d):
    block = 2048
    grid = (BATCH // block,)
    return pl.pallas_call(
        _linear_body,
        grid=grid,
        in_specs=[
            pl.BlockSpec((block, EMB_DIM), lambda i: (i, 0)),
            pl.BlockSpec((EMB_DIM, NUM_CLASS), lambda i: (0, 0)),
            pl.BlockSpec((1, NUM_CLASS), lambda i: (0, 0)),
        ],
        out_specs=pl.BlockSpec((block, NUM_CLASS), lambda i: (i, 0)),
        out_shape=jax.ShapeDtypeStruct((BATCH, NUM_CLASS), jnp.float32),
    )(x, w_t, b2d)


def kernel(node, emb, fc_w, fc_b):
    idx3d = node.astype(jnp.int32).reshape(NUM_WORKERS, N_GROUPS, LANES)
    gathered = _sc_gather(emb, idx3d).reshape(BATCH, EMB_DIM)
    return _tc_linear(gathered, fc_w.T, fc_b.reshape(1, NUM_CLASS))
